# baseline (speedup 1.0000x reference)
import functools
import jax
import jax.numpy as jnp
from jax import lax
from jax.experimental import pallas as pl
from jax.experimental.pallas import tpu as pltpu
from jax.experimental.pallas import tpu_sc as plsc

DIM = 64
B_TOTAL = 16384 * 50
NC, NS = 2, 16
NW = NC * NS
PER_W = B_TOTAL // NW
IPG = 128
GPC = 4
CHUNK = IPG * GPC
NCHUNK = PER_W // CHUNK
IDX_ROWS = PER_W // IPG

@functools.cache
def _build():
    mesh = plsc.VectorSubcoreMesh(core_axis_name="c", subcore_axis_name="s")
    return pl.kernel(
        _emb_lookup,
        mesh=mesh,
        out_type=jax.ShapeDtypeStruct((B_TOTAL, DIM), jnp.float32),
        scratch_types=[
            pltpu.VMEM((IDX_ROWS, IPG), jnp.int32),
            pltpu.VMEM((CHUNK, DIM), jnp.float32),
            pltpu.SemaphoreType.DMA,
        ],
        compiler_params=pltpu.CompilerParams(use_tc_tiling_on_sc=False),
    )


def _emb_lookup(idx_hbm, table_hbm, out_hbm, idx_v, rows_v, sem):
    cid = lax.axis_index("c")
    sid = lax.axis_index("s")
    wid = sid * NC + cid
    row0 = wid * PER_W
    irow0 = wid * IDX_ROWS

    pltpu.sync_copy(idx_hbm.at[pl.ds(irow0, IDX_ROWS)], idx_v)

    def chunk_body(g, carry):
        copies = [
            pltpu.async_copy(
                table_hbm.at[idx_v.at[g * GPC + j]],
                rows_v.at[pl.ds(j * IPG, IPG)],
                sem,
            )
            for j in range(GPC)
        ]
        for cp in copies:
            cp.wait()

        def scale_body(i, c2):
            for j in range(DIM // 16):
                sl = pl.ds(j * 16, 16)
                rows_v[i, sl] = rows_v[i, sl] * 8.0
            return c2

        lax.fori_loop(0, CHUNK, scale_body, 0, unroll=2)
        pltpu.sync_copy(rows_v, out_hbm.at[pl.ds(row0 + g * CHUNK, CHUNK)])
        return carry

    lax.fori_loop(0, NCHUNK, chunk_body, 0)


def kernel(x, table):
    idx = x.reshape(-1).astype(jnp.int32).reshape(NW * IDX_ROWS, IPG)
    out = _build()(idx, table)
    return out.reshape(16384, 50, DIM)
